# Initial kernel scaffold; baseline (speedup 1.0000x reference)
import jax
import jax.numpy as jnp
from jax.experimental import pallas as pl


def kernel(input_sequence, embedding_weight):
    raise NotImplementedError("write your pallas kernel here")


# speedup vs baseline: 4.9491x; 4.9491x over previous
import functools

import jax
import jax.numpy as jnp
from jax import lax
from jax.experimental import pallas as pl
from jax.experimental.pallas import tpu as pltpu
from jax.experimental.pallas import tpu_sc as plsc

VOCAB = 1000000
EMBED_DIM = 32
BATCH = 16384
HIST = 200

NUM_CORES = 2
NUM_SUBCORES = 16
NW = NUM_CORES * NUM_SUBCORES

TOT = BATCH * HIST
IDX_ROW = 128
NROWS = TOT // IDX_ROW
ROWS_PER_W = NROWS // NW
K = 16
CHUNKS = ROWS_PER_W // K
CHUNK_ELEMS = K * IDX_ROW


def _gather_body(idx_hbm, table_hbm, out_hbm, idx_v, rows_v, sem):
    wid = lax.axis_index("s") * NUM_CORES + lax.axis_index("c")
    row_base = wid * ROWS_PER_W

    def chunk(t, carry):
        row0 = row_base + t * K
        pltpu.sync_copy(idx_hbm.at[pl.ds(row0, K)], idx_v)
        copies = []
        for j in range(K):
            copies.append(
                pltpu.async_copy(
                    table_hbm.at[idx_v.at[j]],
                    rows_v.at[pl.ds(j * IDX_ROW, IDX_ROW)],
                    sem,
                )
            )
        for c in copies:
            c.wait()
        pltpu.sync_copy(rows_v, out_hbm.at[pl.ds(row0 * IDX_ROW, CHUNK_ELEMS)])
        return carry

    lax.fori_loop(0, CHUNKS, chunk, 0)


@jax.jit
def _embed_lookup(idx2d, table):
    mesh = plsc.VectorSubcoreMesh(
        core_axis_name="c", subcore_axis_name="s",
        num_cores=NUM_CORES, num_subcores=NUM_SUBCORES,
    )
    fn = pl.kernel(
        _gather_body,
        out_type=jax.ShapeDtypeStruct((TOT, EMBED_DIM), jnp.float32),
        mesh=mesh,
        scratch_types=[
            pltpu.VMEM((K, IDX_ROW), jnp.int32),
            pltpu.VMEM((CHUNK_ELEMS, EMBED_DIM), jnp.float32),
            pltpu.SemaphoreType.DMA,
        ],
        compiler_params=pltpu.CompilerParams(use_tc_tiling_on_sc=False),
    )
    return fn(idx2d, table)


def kernel(input_sequence, embedding_weight):
    idx2d = input_sequence.astype(jnp.int32).reshape(NROWS, IDX_ROW)
    flat = _embed_lookup(idx2d, embedding_weight)
    return flat.reshape(BATCH, HIST, EMBED_DIM)
